# initial kernel scaffold (unmeasured)
import jax
import jax.numpy as jnp
from jax import lax
from jax.experimental import pallas as pl
from jax.experimental.pallas import tpu as pltpu

N_DEV = 4


def kernel(x, w_mat):
    m, k_local = x.shape
    _, n = w_mat.shape
    m_per = m // N_DEV

    def body(x_ref, w_ref, out_ref, comm_ref, send_sems, recv_sems, credit_sem):
        my = lax.axis_index("i")
        left = (my + N_DEV - 1) % N_DEV
        right = (my + 1) % N_DEV

        barrier_sem = pltpu.get_barrier_semaphore()
        for nbr in (left, right):
            pl.semaphore_signal(
                barrier_sem, inc=1,
                device_id=(nbr,), device_id_type=pl.DeviceIdType.MESH,
            )
        pl.semaphore_wait(barrier_sem, 2)

        def partial(c):
            xc = x_ref[pl.ds(c * m_per, m_per), :]
            return jnp.dot(xc, w_ref[:, :], preferred_element_type=jnp.float32)

        comm_ref[0] = partial((my + N_DEV - 1) % N_DEV).astype(jnp.bfloat16)

        for s in range(N_DEV - 1):
            send_slot = s % 2
            recv_slot = (s + 1) % 2
            if s > 0:
                pl.semaphore_wait(credit_sem, 1)
            rdma = pltpu.make_async_remote_copy(
                src_ref=comm_ref.at[send_slot],
                dst_ref=comm_ref.at[recv_slot],
                send_sem=send_sems.at[s],
                recv_sem=recv_sems.at[s],
                device_id=(right,),
                device_id_type=pl.DeviceIdType.MESH,
            )
            rdma.start()
            rdma.wait()
            if s < N_DEV - 2:
                pl.semaphore_signal(
                    credit_sem, inc=1,
                    device_id=(left,), device_id_type=pl.DeviceIdType.MESH,
                )
            c = (my + 2 * N_DEV - 2 - s) % N_DEV
            acc = comm_ref[recv_slot].astype(jnp.float32) + partial(c)
            if s < N_DEV - 2:
                comm_ref[recv_slot] = acc.astype(jnp.bfloat16)
            else:
                out_ref[:, :] = acc

    return pl.pallas_call(
        body,
        out_shape=jax.ShapeDtypeStruct((m_per, n), jnp.float32),
        in_specs=[
            pl.BlockSpec(memory_space=pltpu.VMEM),
            pl.BlockSpec(memory_space=pltpu.VMEM),
        ],
        out_specs=pl.BlockSpec(memory_space=pltpu.VMEM),
        scratch_shapes=[
            pltpu.VMEM((2, m_per, n), jnp.bfloat16),
            pltpu.SemaphoreType.DMA((N_DEV - 1,)),
            pltpu.SemaphoreType.DMA((N_DEV - 1,)),
            pltpu.SemaphoreType.REGULAR,
        ],
        compiler_params=pltpu.CompilerParams(collective_id=0),
    )(x, w_mat)


# baseline (device time: 712452 ns/iter reference)
import jax
import jax.numpy as jnp
from jax import lax
from jax.experimental import pallas as pl
from jax.experimental.pallas import tpu as pltpu

N_DEV = 4
TN = 2048


def kernel(x, w_mat):
    m, k_local = x.shape
    _, n = w_mat.shape
    m_per = m // N_DEV
    n_tiles = n // TN

    xb = x.astype(jnp.bfloat16)
    wb = w_mat.astype(jnp.bfloat16)

    def body(x_ref, w_ref, out_ref, comm_ref, acc_ref,
             send_sem, recv_sem, store_sem, credit_sem):
        my = lax.axis_index("i")
        left = (my + N_DEV - 1) % N_DEV
        right = (my + 1) % N_DEV

        barrier_sem = pltpu.get_barrier_semaphore()
        for nbr in (left, right):
            pl.semaphore_signal(
                barrier_sem, inc=1,
                device_id=(nbr,), device_id_type=pl.DeviceIdType.MESH,
            )
        pl.semaphore_wait(barrier_sem, 2)

        def partial(c, col0):
            xc = x_ref[pl.ds(c * m_per, m_per), :]
            return jnp.dot(xc, w_ref[:, col0:col0 + TN],
                           preferred_element_type=jnp.float32)

        n_hops = n_tiles * (N_DEV - 1)
        hop = 0
        for t in range(n_tiles):
            col0 = t * TN
            comm_ref[0] = partial((my + N_DEV - 1) % N_DEV, col0).astype(jnp.bfloat16)
            for s in range(N_DEV - 1):
                send_slot = s % 2
                recv_slot = (s + 1) % 2
                if hop > 0:
                    pl.semaphore_wait(credit_sem, 1)
                rdma = pltpu.make_async_remote_copy(
                    src_ref=comm_ref.at[send_slot],
                    dst_ref=comm_ref.at[recv_slot],
                    send_sem=send_sem,
                    recv_sem=recv_sem,
                    device_id=(right,),
                    device_id_type=pl.DeviceIdType.MESH,
                )
                rdma.start()
                rdma.wait()
                c = (my + 2 * N_DEV - 2 - s) % N_DEV
                if s < N_DEV - 2:
                    comm_ref[recv_slot] = (
                        comm_ref[recv_slot].astype(jnp.float32) + partial(c, col0)
                    ).astype(jnp.bfloat16)
                else:
                    acc_ref[...] = comm_ref[recv_slot].astype(jnp.float32) + partial(c, col0)
                hop += 1
                if hop < n_hops:
                    pl.semaphore_signal(
                        credit_sem, inc=1,
                        device_id=(left,), device_id_type=pl.DeviceIdType.MESH,
                    )
                if s == N_DEV - 2:
                    store = pltpu.make_async_copy(
                        acc_ref, out_ref.at[:, pl.ds(col0, TN)], store_sem,
                    )
                    store.start()
                    store.wait()

    return pl.pallas_call(
        body,
        out_shape=jax.ShapeDtypeStruct((m_per, n), jnp.float32),
        in_specs=[
            pl.BlockSpec(memory_space=pltpu.VMEM),
            pl.BlockSpec(memory_space=pltpu.VMEM),
        ],
        out_specs=pl.BlockSpec(memory_space=pl.ANY),
        scratch_shapes=[
            pltpu.VMEM((2, m_per, TN), jnp.bfloat16),
            pltpu.VMEM((m_per, TN), jnp.float32),
            pltpu.SemaphoreType.DMA,
            pltpu.SemaphoreType.DMA,
            pltpu.SemaphoreType.DMA,
            pltpu.SemaphoreType.REGULAR,
        ],
        compiler_params=pltpu.CompilerParams(
            collective_id=0,
            vmem_limit_bytes=40 * 1024 * 1024,
        ),
    )(xb, wb)


# device time: 348558 ns/iter; 2.0440x vs baseline; 2.0440x over previous
import jax
import jax.numpy as jnp
from jax import lax
from jax.experimental import pallas as pl
from jax.experimental.pallas import tpu as pltpu

N_DEV = 4
TN = 1024
DEPTH = 2


def kernel(x, w_mat):
    m, k_local = x.shape
    _, n = w_mat.shape
    m_per = m // N_DEV
    n_tiles = n // TN
    per_dir = n_tiles // 2
    n_pairs = per_dir // DEPTH

    xb = x.astype(jnp.bfloat16)
    wb = w_mat.astype(jnp.bfloat16)

    def body(x_ref, w_ref, out_ref, comm_ref, acc_ref,
             send_sems, recv_sems, store_sem, credit_r, credit_l):
        my = lax.axis_index("i")
        left = (my + N_DEV - 1) % N_DEV
        right = (my + 1) % N_DEV

        barrier_sem = pltpu.get_barrier_semaphore()
        for nbr in (left, right):
            pl.semaphore_signal(
                barrier_sem, inc=1,
                device_id=(nbr,), device_id_type=pl.DeviceIdType.MESH,
            )
        pl.semaphore_wait(barrier_sem, 2)

        SUB = TN // 2

        def partial(c, col0, width=TN):
            xc = x_ref[pl.ds(c * m_per, m_per), :]
            return jnp.dot(xc, w_ref[:, col0:col0 + width],
                           preferred_element_type=jnp.float32)

        def desc(d, j, s):
            peer = right if d == 0 else left
            return pltpu.make_async_remote_copy(
                src_ref=comm_ref.at[d, j, s % 2],
                dst_ref=comm_ref.at[d, j, (s + 1) % 2],
                send_sem=send_sems.at[d, j],
                recv_sem=recv_sems.at[d, j],
                device_id=(peer,),
                device_id_type=pl.DeviceIdType.MESH,
            )

        def send_chunk(d, s):
            off = (N_DEV - 1 - s) if d == 0 else (1 + s)
            return (my + off) % N_DEV

        def recv_chunk(d, s):
            off = (2 * N_DEV - 2 - s) if d == 0 else (2 + s)
            return (my + off) % N_DEV

        def credit_sig(d):
            if d == 0:
                pl.semaphore_signal(credit_r, inc=1, device_id=(left,),
                                    device_id_type=pl.DeviceIdType.MESH)
            else:
                pl.semaphore_signal(credit_l, inc=1, device_id=(right,),
                                    device_id_type=pl.DeviceIdType.MESH)

        def credit_wait(d):
            pl.semaphore_wait(credit_r if d == 0 else credit_l, 1)

        def store_desc(col0):
            return pltpu.make_async_copy(
                acc_ref, out_ref.at[:, pl.ds(col0, TN)], store_sem,
            )

        store_col = [None]
        for p in range(n_pairs):
            def col0_of(d, j, p=p):
                t = (0 if d == 0 else per_dir) + p * DEPTH + j
                return t * TN

            for j in range(DEPTH):
                for d in range(2):
                    if p > 0:
                        credit_wait(d)
                    comm_ref[d, j, 0] = partial(
                        send_chunk(d, 0), col0_of(d, j)).astype(jnp.bfloat16)
                    desc(d, j, 0).start()

            for s in (1, 2):
                for j in range(DEPTH):
                    for d in range(2):
                        col0 = col0_of(d, j)
                        c = recv_chunk(d, s - 1)
                        pp0 = partial(c, col0, SUB)
                        desc(d, j, s - 1).wait_recv()
                        comm_ref[d, j, s % 2, :, 0:SUB] = (
                            comm_ref[d, j, s % 2, :, 0:SUB].astype(jnp.float32)
                            + pp0
                        ).astype(jnp.bfloat16)
                        pp1 = partial(c, col0 + SUB, SUB)
                        comm_ref[d, j, s % 2, :, SUB:TN] = (
                            comm_ref[d, j, s % 2, :, SUB:TN].astype(jnp.float32)
                            + pp1
                        ).astype(jnp.bfloat16)
                        desc(d, j, s - 1).wait_send()
                        credit_sig(d)
                        credit_wait(d)
                        desc(d, j, s).start()

            for j in range(DEPTH):
                for d in range(2):
                    col0 = col0_of(d, j)
                    pp0 = partial(my, col0, SUB)
                    desc(d, j, 2).wait_recv()
                    if store_col[0] is not None:
                        store_desc(store_col[0]).wait()
                    acc_ref[:, 0:SUB] = (
                        comm_ref[d, j, 1, :, 0:SUB].astype(jnp.float32) + pp0
                    )
                    acc_ref[:, SUB:TN] = (
                        comm_ref[d, j, 1, :, SUB:TN].astype(jnp.float32)
                        + partial(my, col0 + SUB, SUB)
                    )
                    desc(d, j, 2).wait_send()
                    if p < n_pairs - 1:
                        credit_sig(d)
                    store_desc(col0).start()
                    store_col[0] = col0

        if store_col[0] is not None:
            store_desc(store_col[0]).wait()

    return pl.pallas_call(
        body,
        out_shape=jax.ShapeDtypeStruct((m_per, n), jnp.float32),
        in_specs=[
            pl.BlockSpec(memory_space=pltpu.VMEM),
            pl.BlockSpec(memory_space=pltpu.VMEM),
        ],
        out_specs=pl.BlockSpec(memory_space=pl.ANY),
        scratch_shapes=[
            pltpu.VMEM((2, DEPTH, 2, m_per, TN), jnp.bfloat16),
            pltpu.VMEM((m_per, TN), jnp.float32),
            pltpu.SemaphoreType.DMA((2, DEPTH)),
            pltpu.SemaphoreType.DMA((2, DEPTH)),
            pltpu.SemaphoreType.DMA,
            pltpu.SemaphoreType.REGULAR,
            pltpu.SemaphoreType.REGULAR,
        ],
        compiler_params=pltpu.CompilerParams(
            collective_id=0,
            vmem_limit_bytes=39 * 1024 * 1024,
        ),
    )(xb, wb)


# device time: 338337 ns/iter; 2.1057x vs baseline; 1.0302x over previous
import jax
import jax.numpy as jnp
from jax import lax
from jax.experimental import pallas as pl
from jax.experimental.pallas import tpu as pltpu

N_DEV = 4
TN = 1024
DEPTH = 2


def kernel(x, w_mat):
    m, k_local = x.shape
    _, n = w_mat.shape
    m_per = m // N_DEV
    n_tiles = n // TN
    per_dir = n_tiles // 2
    n_pairs = per_dir // DEPTH

    xb = x.astype(jnp.bfloat16)
    wb = w_mat.astype(jnp.bfloat16)

    def body(x_ref, w_ref, out_ref, comm_ref, acc_ref,
             send_sems, recv_sems, store_sem, credit_r, credit_l):
        my = lax.axis_index("i")
        left = (my + N_DEV - 1) % N_DEV
        right = (my + 1) % N_DEV

        barrier_sem = pltpu.get_barrier_semaphore()
        for nbr in (left, right):
            pl.semaphore_signal(
                barrier_sem, inc=1,
                device_id=(nbr,), device_id_type=pl.DeviceIdType.MESH,
            )
        pl.semaphore_wait(barrier_sem, 2)

        SUB = TN // 2

        def partial(c, col0, width=TN):
            xc = x_ref[pl.ds(c * m_per, m_per), :]
            return jnp.dot(xc, w_ref[:, col0:col0 + width],
                           preferred_element_type=jnp.float32)

        def desc(d, j, s):
            peer = right if d == 0 else left
            return pltpu.make_async_remote_copy(
                src_ref=comm_ref.at[d, j, s % 2],
                dst_ref=comm_ref.at[d, j, (s + 1) % 2],
                send_sem=send_sems.at[d, j],
                recv_sem=recv_sems.at[d, j],
                device_id=(peer,),
                device_id_type=pl.DeviceIdType.MESH,
            )

        def send_chunk(d, s):
            off = (N_DEV - 1 - s) if d == 0 else (1 + s)
            return (my + off) % N_DEV

        def recv_chunk(d, s):
            off = (2 * N_DEV - 2 - s) if d == 0 else (2 + s)
            return (my + off) % N_DEV

        def credit_sig(d):
            if d == 0:
                pl.semaphore_signal(credit_r, inc=1, device_id=(left,),
                                    device_id_type=pl.DeviceIdType.MESH)
            else:
                pl.semaphore_signal(credit_l, inc=1, device_id=(right,),
                                    device_id_type=pl.DeviceIdType.MESH)

        def credit_wait(d):
            pl.semaphore_wait(credit_r if d == 0 else credit_l, 1)

        def store_desc(col0):
            return pltpu.make_async_copy(
                acc_ref, out_ref.at[:, pl.ds(col0, TN)], store_sem,
            )

        def col0_of(p, d, j):
            t = (0 if d == 0 else per_dir) + p * DEPTH + j
            return t * TN

        def fill_item(p, j, d, first):
            if not first:
                credit_wait(d)
            comm_ref[d, j, 0] = partial(
                send_chunk(d, 0), col0_of(p, d, j)).astype(jnp.bfloat16)
            desc(d, j, 0).start()

        store_col = [None]

        def drain_item(p, j, d):
            col0 = col0_of(p, d, j)
            pp0 = partial(my, col0, SUB)
            desc(d, j, 2).wait_recv()
            if store_col[0] is not None:
                store_desc(store_col[0]).wait()
            acc_ref[:, 0:SUB] = (
                comm_ref[d, j, 1, :, 0:SUB].astype(jnp.float32) + pp0
            )
            acc_ref[:, SUB:TN] = (
                comm_ref[d, j, 1, :, SUB:TN].astype(jnp.float32)
                + partial(my, col0 + SUB, SUB)
            )
            desc(d, j, 2).wait_send()
            if p < n_pairs - 1:
                credit_sig(d)
            store_desc(col0).start()
            store_col[0] = col0

        for j in range(DEPTH):
            for d in range(2):
                fill_item(0, j, d, first=True)

        for p in range(n_pairs):
            for s in (1, 2):
                for j in range(DEPTH):
                    for d in range(2):
                        col0 = col0_of(p, d, j)
                        c = recv_chunk(d, s - 1)
                        pp0 = partial(c, col0, SUB)
                        desc(d, j, s - 1).wait_recv()
                        comm_ref[d, j, s % 2, :, 0:SUB] = (
                            comm_ref[d, j, s % 2, :, 0:SUB].astype(jnp.float32)
                            + pp0
                        ).astype(jnp.bfloat16)
                        pp1 = partial(c, col0 + SUB, SUB)
                        comm_ref[d, j, s % 2, :, SUB:TN] = (
                            comm_ref[d, j, s % 2, :, SUB:TN].astype(jnp.float32)
                            + pp1
                        ).astype(jnp.bfloat16)
                        desc(d, j, s - 1).wait_send()
                        credit_sig(d)
                        credit_wait(d)
                        desc(d, j, s).start()

            for j in range(DEPTH):
                for d in range(2):
                    drain_item(p, j, d)
                    if p + 1 < n_pairs:
                        fill_item(p + 1, j, d, first=False)

        if store_col[0] is not None:
            store_desc(store_col[0]).wait()

    return pl.pallas_call(
        body,
        out_shape=jax.ShapeDtypeStruct((m_per, n), jnp.float32),
        in_specs=[
            pl.BlockSpec(memory_space=pltpu.VMEM),
            pl.BlockSpec(memory_space=pltpu.VMEM),
        ],
        out_specs=pl.BlockSpec(memory_space=pl.ANY),
        scratch_shapes=[
            pltpu.VMEM((2, DEPTH, 2, m_per, TN), jnp.bfloat16),
            pltpu.VMEM((m_per, TN), jnp.float32),
            pltpu.SemaphoreType.DMA((2, DEPTH)),
            pltpu.SemaphoreType.DMA((2, DEPTH)),
            pltpu.SemaphoreType.DMA,
            pltpu.SemaphoreType.REGULAR,
            pltpu.SemaphoreType.REGULAR,
        ],
        compiler_params=pltpu.CompilerParams(
            collective_id=0,
            vmem_limit_bytes=39 * 1024 * 1024,
        ),
    )(xb, wb)


# device time: 312073 ns/iter; 2.2830x vs baseline; 1.0842x over previous
import jax
import jax.numpy as jnp
from jax import lax
from jax.experimental import pallas as pl
from jax.experimental.pallas import tpu as pltpu

N_DEV = 4
TN = 1024
DEPTH = 2


def kernel(x, w_mat):
    m, k_local = x.shape
    _, n = w_mat.shape
    m_per = m // N_DEV
    n_tiles = n // TN
    per_dir = n_tiles // 2
    n_pairs = per_dir // DEPTH

    def body(x_ref, w_ref, out_ref, xb_ref, wb_ref, stg_ref, comm_ref, acc_ref,
             conv_sems, send_sems, recv_sems, store_sem, credit_r, credit_l):
        my = lax.axis_index("i")
        left = (my + N_DEV - 1) % N_DEV
        right = (my + 1) % N_DEV

        barrier_sem = pltpu.get_barrier_semaphore()
        for nbr in (left, right):
            pl.semaphore_signal(
                barrier_sem, inc=1,
                device_id=(nbr,), device_id_type=pl.DeviceIdType.MESH,
            )
        pl.semaphore_wait(barrier_sem, 2)

        x_pieces = [(my + 3) % N_DEV, (my + 1) % N_DEV,
                    (my + 2) % N_DEV, my]
        pieces = (
            [("x", x_pieces[0]), ("w", 0), ("x", x_pieces[1]), ("w", per_dir)]
            + [("w", 1), ("w", per_dir + 1)]
            + [("x", x_pieces[2]), ("x", x_pieces[3])]
            + [("w", t) for t in (2, 3, per_dir + 2, per_dir + 3)]
        )

        def conv_dma(i):
            kind, idx = pieces[i]
            if kind == "x":
                src = x_ref.at[pl.ds(idx * m_per, m_per), :]
            else:
                src = w_ref.at[:, pl.ds(idx * TN, TN)]
            return pltpu.make_async_copy(src, stg_ref.at[i % 2],
                                         conv_sems.at[i % 2])

        def conv_finish(i):
            conv_dma(i).wait()
            kind, idx = pieces[i]
            if kind == "x":
                xb_ref[pl.ds(idx * m_per, m_per), :] = (
                    stg_ref[i % 2].astype(jnp.bfloat16))
            else:
                wb_ref[:, idx * TN:(idx + 1) * TN] = (
                    stg_ref[i % 2].astype(jnp.bfloat16))
            if i + 2 < len(pieces):
                conv_dma(i + 2).start()

        conv_dma(0).start()
        conv_dma(1).start()

        SUB = TN // 2

        def partial(c, col0, width=TN):
            xc = xb_ref[pl.ds(c * m_per, m_per), :]
            return jnp.dot(xc, wb_ref[:, col0:col0 + width],
                           preferred_element_type=jnp.float32)

        def desc(d, j, s):
            peer = right if d == 0 else left
            return pltpu.make_async_remote_copy(
                src_ref=comm_ref.at[d, j, s % 2],
                dst_ref=comm_ref.at[d, j, (s + 1) % 2],
                send_sem=send_sems.at[d, j],
                recv_sem=recv_sems.at[d, j],
                device_id=(peer,),
                device_id_type=pl.DeviceIdType.MESH,
            )

        def send_chunk(d, s):
            off = (N_DEV - 1 - s) if d == 0 else (1 + s)
            return (my + off) % N_DEV

        def recv_chunk(d, s):
            off = (2 * N_DEV - 2 - s) if d == 0 else (2 + s)
            return (my + off) % N_DEV

        def credit_sig(d):
            if d == 0:
                pl.semaphore_signal(credit_r, inc=1, device_id=(left,),
                                    device_id_type=pl.DeviceIdType.MESH)
            else:
                pl.semaphore_signal(credit_l, inc=1, device_id=(right,),
                                    device_id_type=pl.DeviceIdType.MESH)

        def credit_wait(d):
            pl.semaphore_wait(credit_r if d == 0 else credit_l, 1)

        def store_desc(col0):
            return pltpu.make_async_copy(
                acc_ref, out_ref.at[:, pl.ds(col0, TN)], store_sem,
            )

        def col0_of(p, d, j):
            t = (0 if d == 0 else per_dir) + p * DEPTH + j
            return t * TN

        def fill_item(p, j, d, first):
            if not first:
                credit_wait(d)
            col0 = col0_of(p, d, j)
            c = send_chunk(d, 0)
            comm_ref[d, j, 0, :, 0:SUB] = partial(c, col0, SUB).astype(jnp.bfloat16)
            comm_ref[d, j, 0, :, SUB:TN] = partial(c, col0 + SUB, SUB).astype(jnp.bfloat16)
            desc(d, j, 0).start()

        store_col = [None]

        def drain_item(p, j, d):
            col0 = col0_of(p, d, j)
            pp0 = partial(my, col0, SUB)
            desc(d, j, 2).wait_recv()
            if store_col[0] is not None:
                store_desc(store_col[0]).wait()
            acc_ref[:, 0:SUB] = (
                comm_ref[d, j, 1, :, 0:SUB].astype(jnp.float32) + pp0
            )
            acc_ref[:, SUB:TN] = (
                comm_ref[d, j, 1, :, SUB:TN].astype(jnp.float32)
                + partial(my, col0 + SUB, SUB)
            )
            desc(d, j, 2).wait_send()
            if p < n_pairs - 1:
                credit_sig(d)
            store_desc(col0).start()
            store_col[0] = col0

        conv_finish(0)
        conv_finish(1)
        fill_item(0, 0, 0, first=True)
        conv_finish(2)
        conv_finish(3)
        fill_item(0, 0, 1, first=True)
        conv_finish(4)
        fill_item(0, 1, 0, first=True)
        conv_finish(5)
        fill_item(0, 1, 1, first=True)
        for i in range(6, len(pieces)):
            conv_finish(i)

        for p in range(n_pairs):
            for s in (1, 2):
                for j in range(DEPTH):
                    for d in range(2):
                        col0 = col0_of(p, d, j)
                        c = recv_chunk(d, s - 1)
                        pp0 = partial(c, col0, SUB)
                        desc(d, j, s - 1).wait_recv()
                        comm_ref[d, j, s % 2, :, 0:SUB] = (
                            comm_ref[d, j, s % 2, :, 0:SUB].astype(jnp.float32)
                            + pp0
                        ).astype(jnp.bfloat16)
                        pp1 = partial(c, col0 + SUB, SUB)
                        comm_ref[d, j, s % 2, :, SUB:TN] = (
                            comm_ref[d, j, s % 2, :, SUB:TN].astype(jnp.float32)
                            + pp1
                        ).astype(jnp.bfloat16)
                        desc(d, j, s - 1).wait_send()
                        credit_sig(d)
                        credit_wait(d)
                        desc(d, j, s).start()

            for j in range(DEPTH):
                for d in range(2):
                    drain_item(p, j, d)
                    if p + 1 < n_pairs:
                        fill_item(p + 1, j, d, first=False)

        if store_col[0] is not None:
            store_desc(store_col[0]).wait()

    return pl.pallas_call(
        body,
        out_shape=jax.ShapeDtypeStruct((m_per, n), jnp.float32),
        in_specs=[
            pl.BlockSpec(memory_space=pl.ANY),
            pl.BlockSpec(memory_space=pl.ANY),
        ],
        out_specs=pl.BlockSpec(memory_space=pl.ANY),
        scratch_shapes=[
            pltpu.VMEM((m, k_local), jnp.bfloat16),
            pltpu.VMEM((k_local, n), jnp.bfloat16),
            pltpu.VMEM((2, m_per, TN), jnp.float32),
            pltpu.VMEM((2, DEPTH, 2, m_per, TN), jnp.bfloat16),
            pltpu.VMEM((m_per, TN), jnp.float32),
            pltpu.SemaphoreType.DMA((2,)),
            pltpu.SemaphoreType.DMA((2, DEPTH)),
            pltpu.SemaphoreType.DMA((2, DEPTH)),
            pltpu.SemaphoreType.DMA,
            pltpu.SemaphoreType.REGULAR,
            pltpu.SemaphoreType.REGULAR,
        ],
        compiler_params=pltpu.CompilerParams(
            collective_id=0,
            vmem_limit_bytes=int(63.5 * 1024 * 1024),
        ),
    )(x, w_mat)


# device time: 311964 ns/iter; 2.2838x vs baseline; 1.0003x over previous
import jax
import jax.numpy as jnp
from jax import lax
from jax.experimental import pallas as pl
from jax.experimental.pallas import tpu as pltpu

N_DEV = 4
TN = 1024
DEPTH = 2


def kernel(x, w_mat):
    m, k_local = x.shape
    _, n = w_mat.shape
    m_per = m // N_DEV
    n_tiles = n // TN
    per_dir = n_tiles // 2
    n_pairs = per_dir // DEPTH

    def body(x_ref, w_ref, out_ref, xb_ref, wb_ref, stg_ref, comm_ref, acc_ref,
             conv_sems, send_sems, recv_sems, store_sem, credit_r, credit_l):
        my = lax.axis_index("i")
        left = (my + N_DEV - 1) % N_DEV
        right = (my + 1) % N_DEV

        barrier_sem = pltpu.get_barrier_semaphore()
        for nbr in (left, right):
            pl.semaphore_signal(
                barrier_sem, inc=1,
                device_id=(nbr,), device_id_type=pl.DeviceIdType.MESH,
            )
        pl.semaphore_wait(barrier_sem, 2)

        x_pieces = [(my + 3) % N_DEV, (my + 1) % N_DEV,
                    (my + 2) % N_DEV, my]
        pieces = (
            [("x", x_pieces[0]), ("w", 0), ("x", x_pieces[1]), ("w", per_dir)]
            + [("w", 1), ("w", per_dir + 1)]
            + [("x", x_pieces[2]), ("x", x_pieces[3])]
            + [("w", t) for t in (2, 3, per_dir + 2, per_dir + 3)]
        )

        def conv_dma(i):
            kind, idx = pieces[i]
            if kind == "x":
                src = x_ref.at[pl.ds(idx * m_per, m_per), :]
            else:
                src = w_ref.at[:, pl.ds(idx * TN, TN)]
            return pltpu.make_async_copy(src, stg_ref.at[i % 2],
                                         conv_sems.at[i % 2])

        def conv_finish(i):
            conv_dma(i).wait()
            kind, idx = pieces[i]
            if kind == "x":
                xb_ref[pl.ds(idx * m_per, m_per), :] = (
                    stg_ref[i % 2].astype(jnp.bfloat16))
            else:
                wb_ref[:, idx * TN:(idx + 1) * TN] = (
                    stg_ref[i % 2].astype(jnp.bfloat16))
            if i + 2 < len(pieces):
                conv_dma(i + 2).start()

        conv_dma(0).start()
        conv_dma(1).start()

        SUB = TN // 2

        def partial(c, col0, width=TN):
            xc = xb_ref[pl.ds(c * m_per, m_per), :]
            return jnp.dot(xc, wb_ref[:, col0:col0 + width],
                           preferred_element_type=jnp.float32)

        def desc(d, j, s):
            peer = right if d == 0 else left
            return pltpu.make_async_remote_copy(
                src_ref=comm_ref.at[d, j, s % 2],
                dst_ref=comm_ref.at[d, j, (s + 1) % 2],
                send_sem=send_sems.at[d, j],
                recv_sem=recv_sems.at[d, j],
                device_id=(peer,),
                device_id_type=pl.DeviceIdType.MESH,
            )

        def send_chunk(d, s):
            off = (N_DEV - 1 - s) if d == 0 else (1 + s)
            return (my + off) % N_DEV

        def recv_chunk(d, s):
            off = (2 * N_DEV - 2 - s) if d == 0 else (2 + s)
            return (my + off) % N_DEV

        def credit_sig(d):
            if d == 0:
                pl.semaphore_signal(credit_r, inc=1, device_id=(left,),
                                    device_id_type=pl.DeviceIdType.MESH)
            else:
                pl.semaphore_signal(credit_l, inc=1, device_id=(right,),
                                    device_id_type=pl.DeviceIdType.MESH)

        def credit_wait(d):
            pl.semaphore_wait(credit_r if d == 0 else credit_l, 1)

        def store_desc(col0):
            return pltpu.make_async_copy(
                acc_ref, out_ref.at[:, pl.ds(col0, TN)], store_sem,
            )

        def col0_of(p, d, j):
            t = (0 if d == 0 else per_dir) + p * DEPTH + j
            return t * TN

        def fill_item(p, j, d, first):
            if not first:
                credit_wait(d)
            col0 = col0_of(p, d, j)
            c = send_chunk(d, 0)
            comm_ref[d, j, 0, :, 0:SUB] = partial(c, col0, SUB).astype(jnp.bfloat16)
            comm_ref[d, j, 0, :, SUB:TN] = partial(c, col0 + SUB, SUB).astype(jnp.bfloat16)
            desc(d, j, 0).start()

        store_col = [None]

        def drain_item(p, j, d):
            col0 = col0_of(p, d, j)
            pp0 = partial(my, col0, SUB)
            desc(d, j, 2).wait_recv()
            if store_col[0] is not None:
                store_desc(store_col[0]).wait()
            acc_ref[:, 0:SUB] = (
                comm_ref[d, j, 1, :, 0:SUB].astype(jnp.float32) + pp0
            )
            acc_ref[:, SUB:TN] = (
                comm_ref[d, j, 1, :, SUB:TN].astype(jnp.float32)
                + partial(my, col0 + SUB, SUB)
            )
            desc(d, j, 2).wait_send()
            if p < n_pairs - 1:
                credit_sig(d)
            store_desc(col0).start()
            store_col[0] = col0

        conv_finish(0)
        conv_finish(1)
        fill_item(0, 0, 0, first=True)
        conv_finish(2)
        conv_finish(3)
        fill_item(0, 0, 1, first=True)
        conv_finish(4)
        fill_item(0, 1, 0, first=True)
        conv_finish(5)
        fill_item(0, 1, 1, first=True)
        for i in range(6, len(pieces)):
            conv_finish(i)

        for p in range(n_pairs):
            for s in (1, 2):
                for j in range(DEPTH):
                    for d in range(2):
                        col0 = col0_of(p, d, j)
                        c = recv_chunk(d, s - 1)
                        pp0 = partial(c, col0, SUB)
                        desc(d, j, s - 1).wait_recv()
                        desc(d, j, s - 1).wait_send()
                        credit_sig(d)
                        comm_ref[d, j, s % 2, :, 0:SUB] = (
                            comm_ref[d, j, s % 2, :, 0:SUB].astype(jnp.float32)
                            + pp0
                        ).astype(jnp.bfloat16)
                        pp1 = partial(c, col0 + SUB, SUB)
                        comm_ref[d, j, s % 2, :, SUB:TN] = (
                            comm_ref[d, j, s % 2, :, SUB:TN].astype(jnp.float32)
                            + pp1
                        ).astype(jnp.bfloat16)
                        credit_wait(d)
                        desc(d, j, s).start()

            for j in range(DEPTH):
                for d in range(2):
                    drain_item(p, j, d)
                    if p + 1 < n_pairs:
                        fill_item(p + 1, j, d, first=False)

        if store_col[0] is not None:
            store_desc(store_col[0]).wait()

    return pl.pallas_call(
        body,
        out_shape=jax.ShapeDtypeStruct((m_per, n), jnp.float32),
        in_specs=[
            pl.BlockSpec(memory_space=pl.ANY),
            pl.BlockSpec(memory_space=pl.ANY),
        ],
        out_specs=pl.BlockSpec(memory_space=pl.ANY),
        scratch_shapes=[
            pltpu.VMEM((m, k_local), jnp.bfloat16),
            pltpu.VMEM((k_local, n), jnp.bfloat16),
            pltpu.VMEM((2, m_per, TN), jnp.float32),
            pltpu.VMEM((2, DEPTH, 2, m_per, TN), jnp.bfloat16),
            pltpu.VMEM((m_per, TN), jnp.float32),
            pltpu.SemaphoreType.DMA((2,)),
            pltpu.SemaphoreType.DMA((2, DEPTH)),
            pltpu.SemaphoreType.DMA((2, DEPTH)),
            pltpu.SemaphoreType.DMA,
            pltpu.SemaphoreType.REGULAR,
            pltpu.SemaphoreType.REGULAR,
        ],
        compiler_params=pltpu.CompilerParams(
            collective_id=0,
            vmem_limit_bytes=int(63.5 * 1024 * 1024),
        ),
    )(x, w_mat)


# device time: 310850 ns/iter; 2.2919x vs baseline; 1.0036x over previous
import jax
import jax.numpy as jnp
from jax import lax
from jax.experimental import pallas as pl
from jax.experimental.pallas import tpu as pltpu

N_DEV = 4
TN = 1024
DEPTH = 2


def kernel(x, w_mat):
    m, k_local = x.shape
    _, n = w_mat.shape
    m_per = m // N_DEV
    n_tiles = n // TN
    per_dir = n_tiles // 2
    n_pairs = per_dir // DEPTH

    def body(x_ref, w_ref, out_ref, xb_ref, wb_ref, stg_ref, comm_ref, acc_ref,
             conv_sems, send_sems, recv_sems, store_sem, credit_r, credit_l):
        my = lax.axis_index("i")
        left = (my + N_DEV - 1) % N_DEV
        right = (my + 1) % N_DEV

        x_pieces = [(my + 3) % N_DEV, (my + 1) % N_DEV,
                    (my + 2) % N_DEV, my]
        pieces = (
            [("x", x_pieces[0]), ("w", 0), ("x", x_pieces[1]), ("w", per_dir)]
            + [("w", 1), ("w", per_dir + 1)]
            + [("x", x_pieces[2]), ("x", x_pieces[3])]
            + [("w", t) for t in (2, 3, per_dir + 2, per_dir + 3)]
        )

        def conv_dma(i):
            kind, idx = pieces[i]
            if kind == "x":
                src = x_ref.at[pl.ds(idx * m_per, m_per), :]
            else:
                src = w_ref.at[:, pl.ds(idx * TN, TN)]
            return pltpu.make_async_copy(src, stg_ref.at[i % 2],
                                         conv_sems.at[i % 2])

        def conv_finish(i):
            conv_dma(i).wait()
            kind, idx = pieces[i]
            if kind == "x":
                xb_ref[pl.ds(idx * m_per, m_per), :] = (
                    stg_ref[i % 2].astype(jnp.bfloat16))
            else:
                wb_ref[:, idx * TN:(idx + 1) * TN] = (
                    stg_ref[i % 2].astype(jnp.bfloat16))
            if i + 2 < len(pieces):
                conv_dma(i + 2).start()

        conv_dma(0).start()
        conv_dma(1).start()

        barrier_sem = pltpu.get_barrier_semaphore()
        for nbr in (left, right):
            pl.semaphore_signal(
                barrier_sem, inc=1,
                device_id=(nbr,), device_id_type=pl.DeviceIdType.MESH,
            )
        pl.semaphore_wait(barrier_sem, 2)

        SUB = TN // 2

        def partial(c, col0, width=TN):
            xc = xb_ref[pl.ds(c * m_per, m_per), :]
            return jnp.dot(xc, wb_ref[:, col0:col0 + width],
                           preferred_element_type=jnp.float32)

        def desc(d, j, s):
            peer = right if d == 0 else left
            return pltpu.make_async_remote_copy(
                src_ref=comm_ref.at[d, j, s % 2],
                dst_ref=comm_ref.at[d, j, (s + 1) % 2],
                send_sem=send_sems.at[d, j],
                recv_sem=recv_sems.at[d, j],
                device_id=(peer,),
                device_id_type=pl.DeviceIdType.MESH,
            )

        def send_chunk(d, s):
            off = (N_DEV - 1 - s) if d == 0 else (1 + s)
            return (my + off) % N_DEV

        def recv_chunk(d, s):
            off = (2 * N_DEV - 2 - s) if d == 0 else (2 + s)
            return (my + off) % N_DEV

        def credit_sig(d):
            if d == 0:
                pl.semaphore_signal(credit_r, inc=1, device_id=(left,),
                                    device_id_type=pl.DeviceIdType.MESH)
            else:
                pl.semaphore_signal(credit_l, inc=1, device_id=(right,),
                                    device_id_type=pl.DeviceIdType.MESH)

        def credit_wait(d):
            pl.semaphore_wait(credit_r if d == 0 else credit_l, 1)

        def store_desc(col0):
            return pltpu.make_async_copy(
                acc_ref, out_ref.at[:, pl.ds(col0, TN)], store_sem,
            )

        def col0_of(p, d, j):
            t = (0 if d == 0 else per_dir) + p * DEPTH + j
            return t * TN

        def fill_item(p, j, d, first):
            col0 = col0_of(p, d, j)
            c = send_chunk(d, 0)
            comm_ref[d, j, 0, :, 0:SUB] = partial(c, col0, SUB).astype(jnp.bfloat16)
            comm_ref[d, j, 0, :, SUB:TN] = partial(c, col0 + SUB, SUB).astype(jnp.bfloat16)
            if not first:
                credit_wait(d)
            desc(d, j, 0).start()

        store_col = [None]

        def drain_item(p, j, d):
            col0 = col0_of(p, d, j)
            pp0 = partial(my, col0, SUB)
            desc(d, j, 2).wait_recv()
            if store_col[0] is not None:
                store_desc(store_col[0]).wait()
            acc_ref[:, 0:SUB] = (
                comm_ref[d, j, 1, :, 0:SUB].astype(jnp.float32) + pp0
            )
            acc_ref[:, SUB:TN] = (
                comm_ref[d, j, 1, :, SUB:TN].astype(jnp.float32)
                + partial(my, col0 + SUB, SUB)
            )
            desc(d, j, 2).wait_send()
            if p < n_pairs - 1:
                credit_sig(d)
            store_desc(col0).start()
            store_col[0] = col0

        conv_finish(0)
        conv_finish(1)
        fill_item(0, 0, 0, first=True)
        conv_finish(2)
        conv_finish(3)
        fill_item(0, 0, 1, first=True)
        conv_finish(4)
        fill_item(0, 1, 0, first=True)
        conv_finish(5)
        fill_item(0, 1, 1, first=True)
        for i in range(6, len(pieces)):
            conv_finish(i)

        for p in range(n_pairs):
            for s in (1, 2):
                for j in range(DEPTH):
                    for d in range(2):
                        col0 = col0_of(p, d, j)
                        c = recv_chunk(d, s - 1)
                        pp0 = partial(c, col0, SUB)
                        desc(d, j, s - 1).wait_recv()
                        desc(d, j, s - 1).wait_send()
                        credit_sig(d)
                        comm_ref[d, j, s % 2, :, 0:SUB] = (
                            comm_ref[d, j, s % 2, :, 0:SUB].astype(jnp.float32)
                            + pp0
                        ).astype(jnp.bfloat16)
                        pp1 = partial(c, col0 + SUB, SUB)
                        comm_ref[d, j, s % 2, :, SUB:TN] = (
                            comm_ref[d, j, s % 2, :, SUB:TN].astype(jnp.float32)
                            + pp1
                        ).astype(jnp.bfloat16)
                        credit_wait(d)
                        desc(d, j, s).start()

            for j in range(DEPTH):
                for d in range(2):
                    drain_item(p, j, d)
                    if p + 1 < n_pairs:
                        fill_item(p + 1, j, d, first=False)

        if store_col[0] is not None:
            store_desc(store_col[0]).wait()

    return pl.pallas_call(
        body,
        out_shape=jax.ShapeDtypeStruct((m_per, n), jnp.float32),
        in_specs=[
            pl.BlockSpec(memory_space=pl.ANY),
            pl.BlockSpec(memory_space=pl.ANY),
        ],
        out_specs=pl.BlockSpec(memory_space=pl.ANY),
        scratch_shapes=[
            pltpu.VMEM((m, k_local), jnp.bfloat16),
            pltpu.VMEM((k_local, n), jnp.bfloat16),
            pltpu.VMEM((2, m_per, TN), jnp.float32),
            pltpu.VMEM((2, DEPTH, 2, m_per, TN), jnp.bfloat16),
            pltpu.VMEM((m_per, TN), jnp.float32),
            pltpu.SemaphoreType.DMA((2,)),
            pltpu.SemaphoreType.DMA((2, DEPTH)),
            pltpu.SemaphoreType.DMA((2, DEPTH)),
            pltpu.SemaphoreType.DMA,
            pltpu.SemaphoreType.REGULAR,
            pltpu.SemaphoreType.REGULAR,
        ],
        compiler_params=pltpu.CompilerParams(
            collective_id=0,
            vmem_limit_bytes=int(63.5 * 1024 * 1024),
        ),
    )(x, w_mat)
